# baseline (speedup 1.0000x reference)
import jax
import jax.numpy as jnp
from jax.experimental import pallas as pl
from jax.experimental.pallas import tpu as pltpu

_B = 4096
_D = 128
_K = 6
_SIGMA = 5
_C = 14.426950408889634
_LN2 = 0.6931471805599453
_LOG_EPS = -18.420680743952367


def _loss_body(f_ref, labr_ref, labc_ref, out_ref):
    f = f_ref[...]
    inv = 1.0 / jnp.maximum(
        jnp.sqrt(jnp.sum(f * f, axis=1, keepdims=True)), 1e-12)
    fall = f * inv
    fbs = fall * _C

    lab_row = labr_ref[...]
    labs = labc_ref[...]

    neg_inf = jnp.float32(-jnp.inf)
    ninf16 = jnp.bfloat16(-jnp.inf)
    lab_row16 = lab_row.astype(jnp.bfloat16)
    labs16 = labs.astype(jnp.bfloat16)
    m1 = jnp.full((_B, 128), ninf16, jnp.bfloat16)
    m2 = jnp.full((_B, 128), ninf16, jnp.bfloat16)
    s_lane = jnp.zeros((_B, 128), jnp.float32)
    one_u = jnp.uint16(1)
    clear_u = jnp.uint16(0xFFFE)
    for c in range(_B // 128):
        fc = fall[c * 128:(c + 1) * 128, :]
        s = jax.lax.dot_general(
            fbs, fc, (((1,), (1,)), ((), ())),
            preferred_element_type=jnp.float32)
        lm = lab_row == labs[:, c * 128:(c + 1) * 128]
        s_lane = s_lane + jnp.exp2(jnp.where(lm, neg_inf, s))
        s16 = s.astype(jnp.bfloat16)
        lm16 = lab_row16 == labs16[:, c * 128:(c + 1) * 128]
        su = jax.lax.bitcast_convert_type(s16, jnp.uint16)
        comp_u = (su & clear_u) | jnp.where(lm16, one_u, jnp.uint16(0))
        comp = jax.lax.bitcast_convert_type(comp_u, jnp.bfloat16)
        m2 = jnp.maximum(m2, jnp.minimum(m1, comp))
        m1 = jnp.maximum(m1, comp)

    neg_sum = jnp.sum(s_lane, axis=1, keepdims=True)

    vals = []
    cnt = jnp.zeros((_B, 1), jnp.float32)
    kk = jnp.float32(_K)
    for k in range(_K):
        m = jnp.max(m1, axis=1, keepdims=True)
        hit = m1 == m
        n = jnp.sum(jnp.where(hit, jnp.bfloat16(1), jnp.bfloat16(0)),
                    axis=1, keepdims=True).astype(jnp.float32)
        m1 = jnp.where(hit, m2, m1)
        m2 = jnp.where(hit, ninf16, m2)
        mu = jax.lax.bitcast_convert_type(m, jnp.uint16)
        matched = (mu & one_u) == one_u
        vclean = jax.lax.bitcast_convert_type(
            mu & clear_u, jnp.bfloat16).astype(jnp.float32)
        c0 = cnt
        c1 = cnt + n
        cnt = c1
        w_neg = jnp.minimum(c1, kk) - jnp.minimum(c0, kk)
        neg_sum = neg_sum - w_neg * jnp.where(matched, 0.0, jnp.exp2(vclean))
        w_pos = jnp.maximum(
            jnp.minimum(c1, kk) - jnp.maximum(c0, 1.0), 0.0)
        vals.append((vclean, w_pos))

    log_ns = jnp.log(neg_sum)
    row_loss = jnp.zeros((_B, 1), jnp.float32)
    for v, w in vals:
        row_loss = row_loss + w * jnp.maximum(v * _LN2 - log_ns, _LOG_EPS)
    total = jnp.sum(row_loss).reshape(1, 1)
    out_ref[...] = jnp.maximum(-total / (_SIGMA * _B), 0.0)


def kernel(features, labels):
    f = features.reshape(_B, _D).astype(jnp.float32)
    labels = labels.astype(jnp.int32)
    lab_col = labels.reshape(1, _B)
    lab_row = labels.reshape(_B, 1)

    out = pl.pallas_call(
        _loss_body,
        out_shape=jax.ShapeDtypeStruct((1, 1), jnp.float32),
    )(f, lab_row, lab_col)

    return out[0, 0]


# speedup vs baseline: 1.0557x; 1.0557x over previous
import jax
import jax.numpy as jnp
from jax.experimental import pallas as pl
from jax.experimental.pallas import tpu as pltpu

_B = 4096
_D = 128
_K = 6
_SIGMA = 5
_C = 14.426950408889634
_LN2 = 0.6931471805599453
_LOG_EPS = -18.420680743952367


def _loss_body(f_ref, labr_ref, labc_ref, out_ref):
    f = f_ref[...]
    inv = 1.0 / jnp.maximum(
        jnp.sqrt(jnp.sum(f * f, axis=1, keepdims=True)), 1e-12)
    fall = f * inv
    fbs = fall * _C

    lab_row = labr_ref[...]
    labs = labc_ref[...]

    neg_inf = jnp.float32(-jnp.inf)
    m1 = jnp.full((_B, 128), neg_inf, jnp.float32)
    m2 = jnp.full((_B, 128), neg_inf, jnp.float32)
    s_lane = jnp.zeros((_B, 128), jnp.float32)
    one_u = jnp.uint32(1)
    clear_u = jnp.uint32(0xFFFFFFFE)
    for c in range(_B // 128):
        fc = fall[c * 128:(c + 1) * 128, :]
        s = jax.lax.dot_general(
            fbs, fc, (((1,), (1,)), ((), ())),
            preferred_element_type=jnp.float32)
        lm = lab_row == labs[:, c * 128:(c + 1) * 128]
        su = jax.lax.bitcast_convert_type(s, jnp.uint32)
        comp_u = (su & clear_u) | jnp.where(lm, one_u, jnp.uint32(0))
        comp = jax.lax.bitcast_convert_type(comp_u, jnp.float32)
        m2 = jnp.maximum(m2, jnp.minimum(m1, comp))
        m1 = jnp.maximum(m1, comp)
        s_lane = s_lane + jnp.exp2(jnp.where(lm, neg_inf, s))

    neg_sum = jnp.sum(s_lane, axis=1, keepdims=True)

    vals = []
    for k in range(_K):
        m = jnp.max(m1, axis=1, keepdims=True)
        hit = m1 == m
        m1 = jnp.where(hit, m2, m1)
        m2 = jnp.where(hit, neg_inf, m2)
        mu = jax.lax.bitcast_convert_type(m, jnp.uint32)
        matched = (mu & one_u) == one_u
        vclean = jax.lax.bitcast_convert_type(mu & clear_u, jnp.float32)
        neg_sum = neg_sum - jnp.where(matched, 0.0, jnp.exp2(vclean))
        if k >= 1:
            vals.append(vclean)

    log_ns = jnp.log(neg_sum)
    row_loss = jnp.zeros((_B, 1), jnp.float32)
    for v in vals:
        row_loss = row_loss + jnp.maximum(v * _LN2 - log_ns, _LOG_EPS)
    total = jnp.sum(row_loss).reshape(1, 1)
    out_ref[...] = jnp.maximum(-total / (_SIGMA * _B), 0.0)


def kernel(features, labels):
    f = features.reshape(_B, _D).astype(jnp.float32)
    labels = labels.astype(jnp.int32)
    lab_col = labels.reshape(1, _B)
    lab_row = labels.reshape(_B, 1)

    out = pl.pallas_call(
        _loss_body,
        out_shape=jax.ShapeDtypeStruct((1, 1), jnp.float32),
    )(f, lab_row, lab_col)

    return out[0, 0]
